# baseline (speedup 1.0000x reference)
import functools

import jax
import jax.numpy as jnp
from jax import lax
from jax.experimental import pallas as pl
from jax.experimental.pallas import tpu as pltpu
from jax.experimental.pallas import tpu_sc as plsc

_NC = 2
_NS = 16
_CHUNK = 128
_FW = 64


def _mesh():
    return plsc.VectorSubcoreMesh(
        core_axis_name="c", subcore_axis_name="s",
        num_cores=_NC, num_subcores=_NS)


@functools.lru_cache(maxsize=None)
def _build_prop(nacc, kch):
    rpt = nacc // _NS
    assert kch % 4 == 0
    kh = kch // 2

    def body(s_hbm, zeros_hbm, gidx_hbm, sidx_hbm, out_hbm,
             gidx_v, sidx_v, rows0, acc_sh, sem0):
        c = lax.axis_index("c")
        s = lax.axis_index("s")
        w = c * _NS + s
        pltpu.sync_copy(gidx_hbm.at[w], gidx_v)
        pltpu.sync_copy(sidx_hbm.at[w], sidx_v)
        pltpu.sync_copy(zeros_hbm.at[pl.ds(s * rpt, rpt)],
                        acc_sh.at[pl.ds(s * rpt, rpt)])
        plsc.subcore_barrier()

        def step(j, carry):
            pltpu.async_copy(s_hbm.at[gidx_v.at[j]], rows0, sem0).wait()
            pltpu.sync_copy(rows0, acc_sh.at[sidx_v.at[j]], add=True)
            return carry

        lax.fori_loop(0, kch, step, 0)
        plsc.subcore_barrier()
        pltpu.sync_copy(acc_sh.at[pl.ds(s * rpt, rpt)],
                        out_hbm.at[pl.ds(c * nacc + s * rpt, rpt)])

    return pl.kernel(
        body,
        out_type=jax.ShapeDtypeStruct((_NC * nacc, _FW), jnp.float32),
        mesh=_mesh(),
        compiler_params=pltpu.CompilerParams(use_tc_tiling_on_sc=False),
        scratch_types=[
            pltpu.VMEM((kch, _CHUNK), jnp.int32),
            pltpu.VMEM((kch, _CHUNK), jnp.int32),
            pltpu.VMEM((_CHUNK, _FW), jnp.float32),
            pltpu.VMEM_SHARED((nacc, _FW), jnp.float32),
            pltpu.SemaphoreType.DMA,
        ],
    )


def _head_body(x_ref, w1_ref, b1_ref, w2_ref, b2_ref, wp_ref, bp_ref, g_ref,
               degp_ref, tx_ref, s_ref, dis_ref, hid_ref, *, rank):
    xb = x_ref[...]
    h1 = lax.dot_general(xb, w1_ref[...], (((1,), (1,)), ((), ())),
                         preferred_element_type=jnp.float32) + b1_ref[...]
    h1 = jnp.maximum(h1, 0.0)
    h2 = lax.dot_general(h1, w2_ref[...], (((1,), (1,)), ((), ())),
                         preferred_element_type=jnp.float32) + b2_ref[...]
    dp = degp_ref[...]
    deg = (dp[0] + dp[1])[:, 0:1]
    dis = jnp.where(deg > 0.0, lax.rsqrt(deg), 0.0)
    dis64 = jnp.broadcast_to(dis, h2.shape)
    tx_ref[...] = h2
    s_ref[...] = dis64 * h2
    dis_ref[...] = dis64
    p0 = jnp.sum(h2 * wp_ref[0:1, :], axis=1, keepdims=True)
    p1 = jnp.sum(h2 * wp_ref[1:2, :], axis=1, keepdims=True)
    hk0 = jnp.tanh(p0 + bp_ref[0, 0])
    hk1 = jnp.tanh(p1 + bp_ref[0, 1])
    eta = (hk0 * g_ref[0, 0] + hk1 * g_ref[0, 1]) * (1.0 / rank)
    hid_ref[...] = h2 * eta


@functools.lru_cache(maxsize=None)
def _build_head(n, f_in, hid, c, rank, bn, nacc):
    grid = (n // bn,)
    full = lambda i: (0, 0)
    return pl.pallas_call(
        functools.partial(_head_body, rank=rank),
        grid=grid,
        in_specs=[
            pl.BlockSpec((bn, f_in), lambda i: (i, 0)),
            pl.BlockSpec((hid, f_in), full),
            pl.BlockSpec((1, hid), full),
            pl.BlockSpec((c, hid), full),
            pl.BlockSpec((1, c), full),
            pl.BlockSpec((rank, c), full),
            pl.BlockSpec((1, rank), full),
            pl.BlockSpec((1, rank), full),
            pl.BlockSpec((2, bn, _FW), lambda i: (0, i, 0)),
        ],
        out_specs=[
            pl.BlockSpec((bn, c), lambda i: (i, 0)),
            pl.BlockSpec((bn, _FW), lambda i: (i, 0)),
            pl.BlockSpec((bn, c), lambda i: (i, 0)),
            pl.BlockSpec((bn, c), lambda i: (i, 0)),
        ],
        out_shape=[
            jax.ShapeDtypeStruct((n, c), jnp.float32),
            jax.ShapeDtypeStruct((n, _FW), jnp.float32),
            jax.ShapeDtypeStruct((n, c), jnp.float32),
            jax.ShapeDtypeStruct((n, c), jnp.float32),
        ],
    )


def _iter_body(p_ref, dis_ref, txold_ref, hid_ref, wp_ref, bp_ref, g_ref,
               tx_ref, s_ref, hidout_ref, *, a, b, rank, final):
    p = p_ref[...]
    q = p[0] + p[1]
    dis = dis_ref[...]
    txn = (-a) * dis * q
    if b != 0.0:
        txn = txn + b * txold_ref[...]
    tx_ref[...] = txn
    s_ref[...] = dis * txn
    p0 = jnp.sum(txn * wp_ref[0:1, :], axis=1, keepdims=True)
    p1 = jnp.sum(txn * wp_ref[1:2, :], axis=1, keepdims=True)
    hk0 = jnp.tanh(p0 + bp_ref[0, 0])
    hk1 = jnp.tanh(p1 + bp_ref[0, 1])
    eta = (hk0 * g_ref[0, 0] + hk1 * g_ref[0, 1]) * (1.0 / rank)
    hid = hid_ref[...] + txn * eta
    if final:
        m = jnp.max(hid, axis=1, keepdims=True)
        z = hid - m
        hidout_ref[...] = z - jnp.log(jnp.sum(jnp.exp(z), axis=1, keepdims=True))
    else:
        hidout_ref[...] = hid


@functools.lru_cache(maxsize=None)
def _build_iter(n, c, rank, bn, nacc, a, b, final):
    grid = (n // bn,)
    full = lambda i: (0, 0)
    return pl.pallas_call(
        functools.partial(_iter_body, a=a, b=b, rank=rank, final=final),
        grid=grid,
        in_specs=[
            pl.BlockSpec((2, bn, c), lambda i: (0, i, 0)),
            pl.BlockSpec((bn, c), lambda i: (i, 0)),
            pl.BlockSpec((bn, c), lambda i: (i, 0)),
            pl.BlockSpec((bn, c), lambda i: (i, 0)),
            pl.BlockSpec((rank, c), full),
            pl.BlockSpec((1, rank), full),
            pl.BlockSpec((1, rank), full),
        ],
        out_specs=[
            pl.BlockSpec((bn, c), lambda i: (i, 0)),
            pl.BlockSpec((bn, _FW), lambda i: (i, 0)),
            pl.BlockSpec((bn, c), lambda i: (i, 0)),
        ],
        out_shape=[
            jax.ShapeDtypeStruct((n, c), jnp.float32),
            jax.ShapeDtypeStruct((n, _FW), jnp.float32),
            jax.ShapeDtypeStruct((n, c), jnp.float32),
        ],
    )


def kernel(x, edge_index, W1, b1, W2, b2, gamma, Wp, bp):
    n, f_in = x.shape
    hid = W1.shape[0]
    c = W2.shape[0]
    rank, kp1 = gamma.shape
    K = kp1 - 1
    e = edge_index.shape[1]
    nw = _NC * _NS
    kch = -(-e // (nw * _CHUNK))
    kch = -(-kch // 4) * 4
    epad = nw * kch * _CHUNK
    nacc = -(-(n + 1) // 128) * 128
    bn = 1000 if n % 1000 == 0 else 500

    row = edge_index[0]
    col = edge_index[1]
    selfm = row == col
    pad = epad - e

    def prep(idx, padval):
        return jnp.concatenate(
            [idx, jnp.full((pad,), padval, jnp.int32)]).reshape(nw, kch, _CHUNK)

    gidx = prep(row, 0)
    sidx = prep(jnp.where(selfm, n, col), n)
    didx = prep(jnp.where(selfm, n, row), n)

    zeros_f = jnp.zeros((nacc, _FW), jnp.float32)
    ones_t = jnp.ones((n, _FW), jnp.float32)

    prop = _build_prop(nacc, kch)

    degp = prop(ones_t, zeros_f, gidx, didx)
    degp = degp.reshape(_NC, nacc, _FW)

    b1r = b1.reshape(1, hid)
    b2r = b2.reshape(1, c)
    head = _build_head(n, f_in, hid, c, rank, bn, nacc)
    tx0, s0, dis64, hidden = head(
        x, W1, b1r, W2, b2r, Wp[0], bp[0:1],
        gamma[:, 0].reshape(1, rank), degp)

    txm2, txm1, s_prev = None, tx0, s0
    for k in range(1, K + 1):
        partial = prop(s_prev, zeros_f, gidx, sidx)
        partial = partial.reshape(_NC, nacc, _FW)
        a, b = (1.0, 0.0) if k == 1 else (2.0, -1.0)
        txold = tx0 if k == 1 else txm2
        fn = _build_iter(n, c, rank, bn, nacc, a, b, k == K)
        txk, sk, hidden = fn(
            partial[:, :n, :c], dis64, txold, hidden, Wp[k], bp[k:k + 1],
            gamma[:, k].reshape(1, rank))
        txm2, txm1, s_prev = txm1, txk, sk
    return hidden


# speedup vs baseline: 1.2303x; 1.2303x over previous
import functools

import jax
import jax.numpy as jnp
from jax import lax
from jax.experimental import pallas as pl
from jax.experimental.pallas import tpu as pltpu
from jax.experimental.pallas import tpu_sc as plsc

_NC = 2
_NS = 16
_CHUNK = 128
_FW = 64


def _mesh():
    return plsc.VectorSubcoreMesh(
        core_axis_name="c", subcore_axis_name="s",
        num_cores=_NC, num_subcores=_NS)


@functools.lru_cache(maxsize=None)
def _build_prop(nacc, kch):
    rpt = nacc // _NS
    assert kch % 4 == 0
    kh = kch // 2

    def body(s_hbm, zeros_hbm, gidx_hbm, sidx_hbm, out_hbm,
             gidx_v, sidx_v, rows0, rows1, rows2, rows3, acc_sh,
             sem0, sem1, sem2, sem3):
        rows = (rows0, rows1, rows2, rows3)
        sems = (sem0, sem1, sem2, sem3)
        c = lax.axis_index("c")
        s = lax.axis_index("s")
        w = c * _NS + s
        pltpu.sync_copy(gidx_hbm.at[w], gidx_v)
        pltpu.sync_copy(sidx_hbm.at[w], sidx_v)
        pltpu.sync_copy(zeros_hbm.at[pl.ds(s * rpt, rpt)],
                        acc_sh.at[pl.ds(s * rpt, rpt)])
        plsc.subcore_barrier()

        for t in range(3):
            pltpu.async_copy(s_hbm.at[gidx_v.at[t]], rows[t], sems[t])

        def step(jj, carry):
            base = 4 * jj
            for t in range(4):
                j = base + t
                jn = lax.rem(j + 3, kch)
                pltpu.async_copy(
                    s_hbm.at[gidx_v.at[jn]], rows[(t + 3) % 4],
                    sems[(t + 3) % 4])
                pltpu.make_async_copy(
                    s_hbm.at[gidx_v.at[j]], rows[t], sems[t]).wait()
                pltpu.sync_copy(rows[t], acc_sh.at[sidx_v.at[j]], add=True)
            return carry

        lax.fori_loop(0, kch // 4, step, 0)
        for t in range(3):
            pltpu.make_async_copy(
                s_hbm.at[gidx_v.at[t]], rows[t], sems[t]).wait()
        plsc.subcore_barrier()
        pltpu.sync_copy(acc_sh.at[pl.ds(s * rpt, rpt)],
                        out_hbm.at[pl.ds(c * nacc + s * rpt, rpt)])

    return pl.kernel(
        body,
        out_type=jax.ShapeDtypeStruct((_NC * nacc, _FW), jnp.float32),
        mesh=_mesh(),
        compiler_params=pltpu.CompilerParams(use_tc_tiling_on_sc=False),
        scratch_types=[
            pltpu.VMEM((kch, _CHUNK), jnp.int32),
            pltpu.VMEM((kch, _CHUNK), jnp.int32),
            pltpu.VMEM((_CHUNK, _FW), jnp.float32),
            pltpu.VMEM((_CHUNK, _FW), jnp.float32),
            pltpu.VMEM((_CHUNK, _FW), jnp.float32),
            pltpu.VMEM((_CHUNK, _FW), jnp.float32),
            pltpu.VMEM_SHARED((nacc, _FW), jnp.float32),
            pltpu.SemaphoreType.DMA,
            pltpu.SemaphoreType.DMA,
            pltpu.SemaphoreType.DMA,
            pltpu.SemaphoreType.DMA,
        ],
    )


def _head_body(x_ref, w1_ref, b1_ref, w2_ref, b2_ref, wp_ref, bp_ref, g_ref,
               degp_ref, tx_ref, s_ref, dis_ref, hid_ref, *, rank):
    xb = x_ref[...]
    h1 = lax.dot_general(xb, w1_ref[...], (((1,), (1,)), ((), ())),
                         preferred_element_type=jnp.float32) + b1_ref[...]
    h1 = jnp.maximum(h1, 0.0)
    h2 = lax.dot_general(h1, w2_ref[...], (((1,), (1,)), ((), ())),
                         preferred_element_type=jnp.float32) + b2_ref[...]
    dp = degp_ref[...]
    deg = (dp[0] + dp[1])[:, 0:1]
    dis = jnp.where(deg > 0.0, lax.rsqrt(deg), 0.0)
    dis64 = jnp.broadcast_to(dis, h2.shape)
    tx_ref[...] = h2
    s_ref[...] = dis64 * h2
    dis_ref[...] = dis64
    p0 = jnp.sum(h2 * wp_ref[0:1, :], axis=1, keepdims=True)
    p1 = jnp.sum(h2 * wp_ref[1:2, :], axis=1, keepdims=True)
    hk0 = jnp.tanh(p0 + bp_ref[0, 0])
    hk1 = jnp.tanh(p1 + bp_ref[0, 1])
    eta = (hk0 * g_ref[0, 0] + hk1 * g_ref[0, 1]) * (1.0 / rank)
    hid_ref[...] = h2 * eta


@functools.lru_cache(maxsize=None)
def _build_head(n, f_in, hid, c, rank, bn, nacc):
    grid = (n // bn,)
    full = lambda i: (0, 0)
    return pl.pallas_call(
        functools.partial(_head_body, rank=rank),
        grid=grid,
        in_specs=[
            pl.BlockSpec((bn, f_in), lambda i: (i, 0)),
            pl.BlockSpec((hid, f_in), full),
            pl.BlockSpec((1, hid), full),
            pl.BlockSpec((c, hid), full),
            pl.BlockSpec((1, c), full),
            pl.BlockSpec((rank, c), full),
            pl.BlockSpec((1, rank), full),
            pl.BlockSpec((1, rank), full),
            pl.BlockSpec((2, bn, _FW), lambda i: (0, i, 0)),
        ],
        out_specs=[
            pl.BlockSpec((bn, c), lambda i: (i, 0)),
            pl.BlockSpec((bn, _FW), lambda i: (i, 0)),
            pl.BlockSpec((bn, c), lambda i: (i, 0)),
            pl.BlockSpec((bn, c), lambda i: (i, 0)),
        ],
        out_shape=[
            jax.ShapeDtypeStruct((n, c), jnp.float32),
            jax.ShapeDtypeStruct((n, _FW), jnp.float32),
            jax.ShapeDtypeStruct((n, c), jnp.float32),
            jax.ShapeDtypeStruct((n, c), jnp.float32),
        ],
    )


def _iter_body(p_ref, dis_ref, txold_ref, hid_ref, wp_ref, bp_ref, g_ref,
               tx_ref, s_ref, hidout_ref, *, a, b, rank, final):
    p = p_ref[...]
    q = p[0] + p[1]
    dis = dis_ref[...]
    txn = (-a) * dis * q
    if b != 0.0:
        txn = txn + b * txold_ref[...]
    tx_ref[...] = txn
    s_ref[...] = dis * txn
    p0 = jnp.sum(txn * wp_ref[0:1, :], axis=1, keepdims=True)
    p1 = jnp.sum(txn * wp_ref[1:2, :], axis=1, keepdims=True)
    hk0 = jnp.tanh(p0 + bp_ref[0, 0])
    hk1 = jnp.tanh(p1 + bp_ref[0, 1])
    eta = (hk0 * g_ref[0, 0] + hk1 * g_ref[0, 1]) * (1.0 / rank)
    hid = hid_ref[...] + txn * eta
    if final:
        m = jnp.max(hid, axis=1, keepdims=True)
        z = hid - m
        hidout_ref[...] = z - jnp.log(jnp.sum(jnp.exp(z), axis=1, keepdims=True))
    else:
        hidout_ref[...] = hid


@functools.lru_cache(maxsize=None)
def _build_iter(n, c, rank, bn, nacc, a, b, final):
    grid = (n // bn,)
    full = lambda i: (0, 0)
    return pl.pallas_call(
        functools.partial(_iter_body, a=a, b=b, rank=rank, final=final),
        grid=grid,
        in_specs=[
            pl.BlockSpec((2, bn, c), lambda i: (0, i, 0)),
            pl.BlockSpec((bn, c), lambda i: (i, 0)),
            pl.BlockSpec((bn, c), lambda i: (i, 0)),
            pl.BlockSpec((bn, c), lambda i: (i, 0)),
            pl.BlockSpec((rank, c), full),
            pl.BlockSpec((1, rank), full),
            pl.BlockSpec((1, rank), full),
        ],
        out_specs=[
            pl.BlockSpec((bn, c), lambda i: (i, 0)),
            pl.BlockSpec((bn, _FW), lambda i: (i, 0)),
            pl.BlockSpec((bn, c), lambda i: (i, 0)),
        ],
        out_shape=[
            jax.ShapeDtypeStruct((n, c), jnp.float32),
            jax.ShapeDtypeStruct((n, _FW), jnp.float32),
            jax.ShapeDtypeStruct((n, c), jnp.float32),
        ],
    )


def kernel(x, edge_index, W1, b1, W2, b2, gamma, Wp, bp):
    n, f_in = x.shape
    hid = W1.shape[0]
    c = W2.shape[0]
    rank, kp1 = gamma.shape
    K = kp1 - 1
    e = edge_index.shape[1]
    nw = _NC * _NS
    kch = -(-e // (nw * _CHUNK))
    kch = -(-kch // 4) * 4
    epad = nw * kch * _CHUNK
    nacc = -(-(n + 1) // 128) * 128
    bn = 1000 if n % 1000 == 0 else 500

    row = edge_index[0]
    col = edge_index[1]
    selfm = row == col
    pad = epad - e

    def prep(idx, padval):
        return jnp.concatenate(
            [idx, jnp.full((pad,), padval, jnp.int32)]).reshape(nw, kch, _CHUNK)

    gidx = prep(row, 0)
    sidx = prep(jnp.where(selfm, n, col), n)
    didx = prep(jnp.where(selfm, n, row), n)

    zeros_f = jnp.zeros((nacc, _FW), jnp.float32)
    ones_t = jnp.ones((n, _FW), jnp.float32)

    prop = _build_prop(nacc, kch)

    degp = prop(ones_t, zeros_f, gidx, didx)
    degp = degp.reshape(_NC, nacc, _FW)

    b1r = b1.reshape(1, hid)
    b2r = b2.reshape(1, c)
    head = _build_head(n, f_in, hid, c, rank, bn, nacc)
    tx0, s0, dis64, hidden = head(
        x, W1, b1r, W2, b2r, Wp[0], bp[0:1],
        gamma[:, 0].reshape(1, rank), degp)

    txm2, txm1, s_prev = None, tx0, s0
    for k in range(1, K + 1):
        partial = prop(s_prev, zeros_f, gidx, sidx)
        partial = partial.reshape(_NC, nacc, _FW)
        a, b = (1.0, 0.0) if k == 1 else (2.0, -1.0)
        txold = tx0 if k == 1 else txm2
        fn = _build_iter(n, c, rank, bn, nacc, a, b, k == K)
        txk, sk, hidden = fn(
            partial[:, :n, :c], dis64, txold, hidden, Wp[k], bp[k:k + 1],
            gamma[:, k].reshape(1, rank))
        txm2, txm1, s_prev = txm1, txk, sk
    return hidden
